# baseline (speedup 1.0000x reference)
import jax
import jax.numpy as jnp
from jax import lax
from jax.experimental import pallas as pl
from jax.experimental.pallas import tpu as pltpu
from jax.experimental.pallas import tpu_sc as plsc

_N = 3
_RG = 8
_CW = 6400
_NSTRIPE = 4


def _body_fn(rows, seq, vocab, valid_hbm, tokens_hbm, lprobs_hbm, out_hbm,
             tok_v, buf_a, buf_b, vld_v, sem_t, sem_la, sem_lb,
             sem_sa, sem_sb):
    npos = seq - _N + 1
    nmatch = (npos + 15) // 16
    n_chunks = (vocab + _CW - 1) // _CW
    tail_off = (n_chunks - 1) * _CW
    tail_w = vocab - tail_off
    per_stripe = n_chunks // _NSTRIPE

    wid = lax.axis_index("c") * 16 + lax.axis_index("s")
    rg = wid // _NSTRIPE
    cs = wid % _NSTRIPE
    row0 = pl.multiple_of(rg * _RG, _RG)
    is_tail = cs == _NSTRIPE - 1

    pltpu.sync_copy(valid_hbm, vld_v)
    valid = vld_v[pl.ds(0, 16)][0] != 0
    lanes = lax.iota(jnp.int32, 16)
    neg_inf = jnp.full((16,), -jnp.inf, dtype=jnp.float32)

    cp_t = pltpu.make_async_copy(tokens_hbm.at[pl.ds(row0, _RG)], tok_v,
                                 sem_t)
    cp_t.start()

    def chunk_off(jj):
        return pl.multiple_of((cs * per_stripe + jj) * _CW, 128)

    def load_c(jj, buf, sem, w=_CW, off=None):
        off = chunk_off(jj) if off is None else off
        return pltpu.make_async_copy(
            lprobs_hbm.at[pl.ds(row0, _RG), pl.ds(off, w)],
            buf.at[:, pl.ds(0, w)], sem)

    def store_c(jj, buf, sem, w=_CW, off=None):
        off = chunk_off(jj) if off is None else off
        return pltpu.make_async_copy(
            buf.at[:, pl.ds(0, w)],
            out_hbm.at[pl.ds(row0, _RG), pl.ds(off, w)], sem)

    ld0 = load_c(0, buf_a, sem_la)
    ld0.start()
    ld1 = load_c(1, buf_b, sem_lb)
    ld1.start()

    cp_t.wait()

    lasts, anys = [], []
    for r in range(_RG):
        tail = tok_v[r, pl.ds(seq - 16, 16)]
        l0, l1 = tail[14], tail[15]
        lasts.append((l0, l1))
        rvec = jnp.full((16,), r, jnp.int32)

        def mbody(i, acc, rvec=rvec, l0=l0, l1=l1):
            idx = lanes + i * 16
            t0 = plsc.load_gather(tok_v, [rvec, jnp.minimum(idx, seq - 1)])
            t1 = plsc.load_gather(tok_v, [rvec, jnp.minimum(idx + 1,
                                                            seq - 1)])
            return acc | ((idx < npos) & (t0 == l0) & (t1 == l1))

        acc = lax.fori_loop(0, nmatch, mbody, jnp.zeros((16,), jnp.bool_))
        anys.append(jnp.any(acc) & valid)

    def scatter(buf, off, w):
        for r in range(_RG):
            l0, l1 = lasts[r]
            rvec = jnp.full((16,), r, jnp.int32)

            @pl.when(anys[r])
            def _(rvec=rvec, l0=l0, l1=l1):
                def sbody(i, c):
                    idx = lanes + i * 16
                    t0 = plsc.load_gather(
                        tok_v, [rvec, jnp.minimum(idx, seq - 1)])
                    t1 = plsc.load_gather(
                        tok_v, [rvec, jnp.minimum(idx + 1, seq - 1)])
                    t2 = plsc.load_gather(
                        tok_v, [rvec, jnp.minimum(idx + 2, seq - 1)])
                    m = ((idx < npos) & (t0 == l0) & (t1 == l1)
                         & (t2 >= off) & (t2 < off + w))
                    plsc.store_scatter(
                        buf, [rvec, jnp.where(m, t2 - off, 0)],
                        neg_inf, mask=m)
                    return c

                lax.fori_loop(0, nmatch, sbody, 0)

    ld3_tail = load_c(3, buf_b, sem_lb, w=tail_w, off=tail_off)
    ld3_full = load_c(3, buf_b, sem_lb)
    st3_tail = store_c(3, buf_b, sem_sb, w=tail_w, off=tail_off)
    st3_full = store_c(3, buf_b, sem_sb)

    ld0.wait()
    scatter(buf_a, chunk_off(0), _CW)
    st0 = store_c(0, buf_a, sem_sa)
    st0.start()

    ld1.wait()
    scatter(buf_b, chunk_off(1), _CW)
    st1 = store_c(1, buf_b, sem_sb)
    st1.start()
    st0.wait()
    ld2 = load_c(2, buf_a, sem_la)
    ld2.start()

    ld2.wait()
    scatter(buf_a, chunk_off(2), _CW)
    st2 = store_c(2, buf_a, sem_sa)
    st2.start()
    st1.wait()

    @pl.when(is_tail)
    def _():
        ld3_tail.start()

    @pl.when(~is_tail)
    def _():
        ld3_full.start()

    @pl.when(is_tail)
    def _():
        ld3_tail.wait()
        scatter(buf_b, tail_off, tail_w)
        st3_tail.start()
        st3_tail.wait()

    @pl.when(~is_tail)
    def _():
        ld3_full.wait()
        scatter(buf_b, chunk_off(3), _CW)
        st3_full.start()
        st3_full.wait()

    st2.wait()


def kernel(tokens, lprobs, bsz, step, beam_size, no_repeat_ngram_size):
    rows, seq = tokens.shape
    vocab = lprobs.shape[1]
    valid = (
        (rows == bsz * beam_size)
        & (step == seq - 1)
        & (no_repeat_ngram_size == _N)
    )
    valid_arr = jnp.full((16,), 0, dtype=jnp.int32) + valid.astype(jnp.int32)

    mesh = plsc.VectorSubcoreMesh(core_axis_name="c", subcore_axis_name="s")

    def body(valid_hbm, tokens_hbm, lprobs_hbm, out_hbm, tok_v, buf_a,
             buf_b, vld_v, sem_t, sem_la, sem_lb, sem_sa, sem_sb):
        _body_fn(rows, seq, vocab, valid_hbm, tokens_hbm, lprobs_hbm,
                 out_hbm, tok_v, buf_a, buf_b, vld_v, sem_t, sem_la,
                 sem_lb, sem_sa, sem_sb)

    run = pl.kernel(
        body,
        out_type=jax.ShapeDtypeStruct((rows, vocab), jnp.float32),
        mesh=mesh,
        compiler_params=pltpu.CompilerParams(
            needs_layout_passes=False,
            skip_device_barrier=True,
            use_tc_tiling_on_sc=False,
        ),
        scratch_types=[
            pltpu.VMEM((_RG, seq), jnp.int32),
            pltpu.VMEM((_RG, _CW), jnp.float32),
            pltpu.VMEM((_RG, _CW), jnp.float32),
            pltpu.VMEM((16,), jnp.int32),
            pltpu.SemaphoreType.DMA,
            pltpu.SemaphoreType.DMA,
            pltpu.SemaphoreType.DMA,
            pltpu.SemaphoreType.DMA,
            pltpu.SemaphoreType.DMA,
        ],
    )
    return run(valid_arr, tokens, lprobs)


# speedup vs baseline: 2.5091x; 2.5091x over previous
import jax
import jax.numpy as jnp
from jax import lax
from jax.experimental import pallas as pl
from jax.experimental.pallas import tpu as pltpu
from jax.experimental.pallas import tpu_sc as plsc

_N = 3
_RG = 8
_CW = 5376
_NSTRIPE = 4


def _body_fn(rows, seq, vocab, valid_hbm, tokens_hbm, lprobs_hbm, out_hbm,
             tok_v, buf_a, buf_b, tail_v, vld_v,
             sem_t, sem_la, sem_lb, sem_sa, sem_sb, sem_tl, sem_ts):
    npos = seq - _N + 1
    nmatch = (npos + 15) // 16
    n_main = 18
    tail_off = n_main * _CW
    tail_w = vocab - tail_off
    stripe_ids = [list(range(0, 5)), list(range(5, 10)),
                  list(range(10, 14)), list(range(14, 18))]
    first_table = [ids[0] for ids in stripe_ids]

    wid = lax.axis_index("c") * 16 + lax.axis_index("s")
    rg = wid // _NSTRIPE
    cs = wid % _NSTRIPE
    row0 = pl.multiple_of(rg * _RG, _RG)

    pltpu.sync_copy(valid_hbm, vld_v)
    valid = vld_v[pl.ds(0, 16)][0] != 0
    lanes = lax.iota(jnp.int32, 16)
    neg_inf = jnp.full((16,), -jnp.inf, dtype=jnp.float32)

    cp_t = pltpu.make_async_copy(tokens_hbm.at[pl.ds(row0, _RG)], tok_v,
                                 sem_t)
    cp_t.start()

    def load_c(off, buf, sem, w=_CW):
        return pltpu.make_async_copy(
            lprobs_hbm.at[pl.ds(row0, _RG), pl.ds(off, w)],
            buf, sem)

    def store_c(off, buf, sem, w=_CW):
        return pltpu.make_async_copy(
            buf, out_hbm.at[pl.ds(row0, _RG), pl.ds(off, w)], sem)

    first = (jnp.where(cs == 0, first_table[0],
             jnp.where(cs == 1, first_table[1],
             jnp.where(cs == 2, first_table[2], first_table[3])))
             ).astype(jnp.int32)
    off0 = pl.multiple_of(first * _CW, 128)
    off1 = pl.multiple_of((first + 1) * _CW, 128)
    load_c(off0, buf_a, sem_la).start()
    load_c(off1, buf_b, sem_lb).start()

    ld_tail = load_c(tail_off, tail_v, sem_tl, w=tail_w)
    st_tail = store_c(tail_off, tail_v, sem_ts, w=tail_w)

    @pl.when(cs == _NSTRIPE - 1)
    def _():
        ld_tail.start()

    cp_t.wait()

    def row_match(r, bits):
        rvec = jnp.full((16,), 0, jnp.int32) + r
        l0 = plsc.load_gather(
            tok_v, [rvec, jnp.full((16,), seq - 2, jnp.int32)])[0]
        l1 = plsc.load_gather(
            tok_v, [rvec, jnp.full((16,), seq - 1, jnp.int32)])[0]

        def mbody(i, acc):
            idx = lanes + i * 16
            t0 = plsc.load_gather(tok_v, [rvec, jnp.minimum(idx, seq - 1)])
            t1 = plsc.load_gather(tok_v, [rvec, jnp.minimum(idx + 1,
                                                            seq - 1)])
            return acc | ((idx < npos) & (t0 == l0) & (t1 == l1))

        acc = lax.fori_loop(0, nmatch, mbody, jnp.zeros((16,), jnp.bool_))
        return bits | (jnp.any(acc).astype(jnp.int32) << r)

    anybits = lax.fori_loop(0, _RG, row_match, jnp.int32(0))
    anybits = jnp.where(valid, anybits, 0)

    def scatter(buf, off, w):
        @pl.when(anybits != 0)
        def _():
            def rloop(r, c):
                rvec = jnp.full((16,), 0, jnp.int32) + r

                @pl.when(((anybits >> r) & 1) != 0)
                def _():
                    l0 = plsc.load_gather(
                        tok_v,
                        [rvec, jnp.full((16,), seq - 2, jnp.int32)])[0]
                    l1 = plsc.load_gather(
                        tok_v,
                        [rvec, jnp.full((16,), seq - 1, jnp.int32)])[0]

                    def sbody(i, c2):
                        idx = lanes + i * 16
                        t0 = plsc.load_gather(
                            tok_v, [rvec, jnp.minimum(idx, seq - 1)])
                        t1 = plsc.load_gather(
                            tok_v, [rvec, jnp.minimum(idx + 1, seq - 1)])
                        t2 = plsc.load_gather(
                            tok_v, [rvec, jnp.minimum(idx + 2, seq - 1)])
                        m = ((idx < npos) & (t0 == l0) & (t1 == l1)
                             & (t2 >= off) & (t2 < off + w))
                        plsc.store_scatter(
                            buf, [rvec, jnp.where(m, t2 - off, 0)],
                            neg_inf, mask=m)
                        return c2

                    lax.fori_loop(0, nmatch, sbody, 0)

                return c

            lax.fori_loop(0, _RG, rloop, 0)

    def stripe_pipeline(ids, has_tail):
        n = len(ids)
        lds = {
            0: load_c(ids[0] * _CW, buf_a, sem_la),
            1: load_c(ids[1] * _CW, buf_b, sem_lb),
        }
        sts = {}
        for jj in range(n):
            buf = buf_a if jj % 2 == 0 else buf_b
            sem_s = sem_sa if jj % 2 == 0 else sem_sb
            lds[jj].wait()
            scatter(buf, ids[jj] * _CW, _CW)
            st = store_c(ids[jj] * _CW, buf, sem_s)
            st.start()
            sts[jj] = st
            if jj >= 1 and jj + 1 < n:
                sts[jj - 1].wait()
                nbuf = buf_a if (jj + 1) % 2 == 0 else buf_b
                sem_l = sem_la if (jj + 1) % 2 == 0 else sem_lb
                nld = load_c(ids[jj + 1] * _CW, nbuf, sem_l)
                nld.start()
                lds[jj + 1] = nld
        if has_tail:
            ld_tail.wait()
            scatter(tail_v, tail_off, tail_w)
            st_tail.start()
            st_tail.wait()
        if n >= 2:
            sts[n - 2].wait()
        sts[n - 1].wait()

    branches = []
    for c in range(_NSTRIPE):
        branches.append(lambda ids=stripe_ids[c], t=(c == _NSTRIPE - 1):
                        stripe_pipeline(ids, t))
    lax.switch(cs, branches)


def kernel(tokens, lprobs, bsz, step, beam_size, no_repeat_ngram_size):
    rows, seq = tokens.shape
    vocab = lprobs.shape[1]
    valid = (
        (rows == bsz * beam_size)
        & (step == seq - 1)
        & (no_repeat_ngram_size == _N)
    )
    valid_arr = jnp.full((16,), 0, dtype=jnp.int32) + valid.astype(jnp.int32)

    mesh = plsc.VectorSubcoreMesh(core_axis_name="c", subcore_axis_name="s")

    def body(valid_hbm, tokens_hbm, lprobs_hbm, out_hbm, tok_v, buf_a,
             buf_b, tail_v, vld_v, sem_t, sem_la, sem_lb, sem_sa, sem_sb,
             sem_tl, sem_ts):
        _body_fn(rows, seq, vocab, valid_hbm, tokens_hbm, lprobs_hbm,
                 out_hbm, tok_v, buf_a, buf_b, tail_v, vld_v,
                 sem_t, sem_la, sem_lb, sem_sa, sem_sb, sem_tl, sem_ts)

    run = pl.kernel(
        body,
        out_type=jax.ShapeDtypeStruct((rows, vocab), jnp.float32),
        mesh=mesh,
        compiler_params=pltpu.CompilerParams(
            needs_layout_passes=False,
            skip_device_barrier=True,
        ),
        scratch_types=[
            pltpu.VMEM((_RG, seq), jnp.int32),
            pltpu.VMEM((_RG, _CW), jnp.float32),
            pltpu.VMEM((_RG, _CW), jnp.float32),
            pltpu.VMEM((_RG, 3232), jnp.float32),
            pltpu.VMEM((16,), jnp.int32),
            pltpu.SemaphoreType.DMA,
            pltpu.SemaphoreType.DMA,
            pltpu.SemaphoreType.DMA,
            pltpu.SemaphoreType.DMA,
            pltpu.SemaphoreType.DMA,
            pltpu.SemaphoreType.DMA,
            pltpu.SemaphoreType.DMA,
        ],
    )
    return run(valid_arr, tokens, lprobs)


# speedup vs baseline: 2.5312x; 1.0088x over previous
import jax
import jax.numpy as jnp
from jax import lax
from jax.experimental import pallas as pl
from jax.experimental.pallas import tpu as pltpu
from jax.experimental.pallas import tpu_sc as plsc

_N = 3
_RG = 8
_CW = 4608
_NSTRIPE = 4
_MT = 32


def _body_fn(rows, seq, vocab, valid_hbm, tokens_hbm, lprobs_hbm, out_hbm,
             tok_v, bufs, mt_v, vld_v, sem_t, sems_l, sems_s, sem_mt):
    npos = seq - _N + 1
    nmatch = (npos + 15) // 16
    mt_off = (vocab // 128) * 128
    starts = [0, 25088, 50048, 75008]
    ends = [25088, 50048, 75008, 99968]
    stripe_chunks = []
    for c in range(_NSTRIPE):
        ch, off = [], starts[c]
        while off < ends[c]:
            w = min(_CW, ends[c] - off)
            ch.append((off, w))
            off += w
        stripe_chunks.append(ch)

    wid = lax.axis_index("c") * 16 + lax.axis_index("s")
    rg = wid // _NSTRIPE
    cs = wid % _NSTRIPE
    row0 = pl.multiple_of(rg * _RG, _RG)

    pltpu.sync_copy(valid_hbm, vld_v)
    valid = vld_v[pl.ds(0, 16)][0] != 0
    lanes = lax.iota(jnp.int32, 16)
    neg_inf = jnp.full((16,), -jnp.inf, dtype=jnp.float32)

    cp_t = pltpu.make_async_copy(tokens_hbm.at[pl.ds(row0, _RG)], tok_v,
                                 sem_t)
    cp_t.start()

    def load_c(off, buf, sem, w=_CW):
        dst = buf if w == buf.shape[1] else buf.at[:, pl.ds(0, w)]
        return pltpu.make_async_copy(
            lprobs_hbm.at[pl.ds(row0, _RG), pl.ds(off, w)], dst, sem)

    def store_c(off, buf, sem, w=_CW):
        src = buf if w == buf.shape[1] else buf.at[:, pl.ds(0, w)]
        return pltpu.make_async_copy(
            src, out_hbm.at[pl.ds(row0, _RG), pl.ds(off, w)], sem)

    start_col = (jnp.where(cs == 0, starts[0],
                 jnp.where(cs == 1, starts[1],
                 jnp.where(cs == 2, starts[2], starts[3])))
                 ).astype(jnp.int32)
    for b in range(3):
        off_b = pl.multiple_of(start_col + b * _CW, 128)
        load_c(off_b, bufs[b], sems_l[b]).start()

    ld_mt = pltpu.make_async_copy(
        lprobs_hbm.at[pl.ds(row0, _RG), pl.ds(mt_off, _MT)], mt_v, sem_mt)
    st_mt = pltpu.make_async_copy(
        mt_v, out_hbm.at[pl.ds(row0, _RG), pl.ds(mt_off, _MT)], sem_mt)

    @pl.when(cs == _NSTRIPE - 1)
    def _():
        ld_mt.start()

    cp_t.wait()

    def row_match(r, bits):
        rvec = jnp.full((16,), 0, jnp.int32) + r
        l0 = plsc.load_gather(
            tok_v, [rvec, jnp.full((16,), seq - 2, jnp.int32)])[0]
        l1 = plsc.load_gather(
            tok_v, [rvec, jnp.full((16,), seq - 1, jnp.int32)])[0]

        def mbody(i, acc):
            idx = lanes + i * 16
            t0 = plsc.load_gather(tok_v, [rvec, jnp.minimum(idx, seq - 1)])
            t1 = plsc.load_gather(tok_v, [rvec, jnp.minimum(idx + 1,
                                                            seq - 1)])
            return acc | ((idx < npos) & (t0 == l0) & (t1 == l1))

        acc = lax.fori_loop(0, nmatch, mbody, jnp.zeros((16,), jnp.bool_))
        return bits | (jnp.any(acc).astype(jnp.int32) << r)

    anybits = lax.fori_loop(0, _RG, row_match, jnp.int32(0))
    anybits = jnp.where(valid, anybits, 0)

    def scatter(buf, off, w):
        @pl.when(anybits != 0)
        def _():
            def rloop(r, c):
                rvec = jnp.full((16,), 0, jnp.int32) + r

                @pl.when(((anybits >> r) & 1) != 0)
                def _():
                    l0 = plsc.load_gather(
                        tok_v,
                        [rvec, jnp.full((16,), seq - 2, jnp.int32)])[0]
                    l1 = plsc.load_gather(
                        tok_v,
                        [rvec, jnp.full((16,), seq - 1, jnp.int32)])[0]

                    def sbody(i, c2):
                        idx = lanes + i * 16
                        t0 = plsc.load_gather(
                            tok_v, [rvec, jnp.minimum(idx, seq - 1)])
                        t1 = plsc.load_gather(
                            tok_v, [rvec, jnp.minimum(idx + 1, seq - 1)])
                        t2 = plsc.load_gather(
                            tok_v, [rvec, jnp.minimum(idx + 2, seq - 1)])
                        m = ((idx < npos) & (t0 == l0) & (t1 == l1)
                             & (t2 >= off) & (t2 < off + w))
                        plsc.store_scatter(
                            buf, [rvec, jnp.where(m, t2 - off, 0)],
                            neg_inf, mask=m)
                        return c2

                    lax.fori_loop(0, nmatch, sbody, 0)

                return c

            lax.fori_loop(0, _RG, rloop, 0)

    def stripe_pipeline(chunks, has_mt):
        n = len(chunks)
        lds, sts = {}, {}
        for b in range(3):
            off, w = chunks[b]
            lds[b] = load_c(off, bufs[b], sems_l[b], w=w)
        for jj in range(n):
            b = jj % 3
            off, w = chunks[jj]
            lds[jj].wait()
            scatter(bufs[b], off, w)
            st = store_c(off, bufs[b], sems_s[b], w=w)
            st.start()
            sts[jj] = st
            if jj >= 2 and jj + 1 < n:
                sts[jj - 2].wait()
                noff, nw = chunks[jj + 1]
                nld = load_c(noff, bufs[(jj + 1) % 3],
                             sems_l[(jj + 1) % 3], w=nw)
                nld.start()
                lds[jj + 1] = nld
        if has_mt:
            ld_mt.wait()
            scatter(mt_v, mt_off, _MT)
            st_mt.start()
            st_mt.wait()
        for jj in range(max(0, n - 2), n):
            sts[jj].wait()

    branches = []
    for c in range(_NSTRIPE):
        branches.append(lambda ch=stripe_chunks[c], t=(c == _NSTRIPE - 1):
                        stripe_pipeline(ch, t))
    lax.switch(cs, branches)


def kernel(tokens, lprobs, bsz, step, beam_size, no_repeat_ngram_size):
    rows, seq = tokens.shape
    vocab = lprobs.shape[1]
    valid = (
        (rows == bsz * beam_size)
        & (step == seq - 1)
        & (no_repeat_ngram_size == _N)
    )
    valid_arr = jnp.full((16,), 0, dtype=jnp.int32) + valid.astype(jnp.int32)

    mesh = plsc.VectorSubcoreMesh(core_axis_name="c", subcore_axis_name="s")

    def body(valid_hbm, tokens_hbm, lprobs_hbm, out_hbm, tok_v, buf_0,
             buf_1, buf_2, mt_v, vld_v, sem_t, sem_l0, sem_l1, sem_l2,
             sem_s0, sem_s1, sem_s2, sem_mt):
        _body_fn(rows, seq, vocab, valid_hbm, tokens_hbm, lprobs_hbm,
                 out_hbm, tok_v, [buf_0, buf_1, buf_2], mt_v, vld_v,
                 sem_t, [sem_l0, sem_l1, sem_l2],
                 [sem_s0, sem_s1, sem_s2], sem_mt)

    run = pl.kernel(
        body,
        out_type=jax.ShapeDtypeStruct((rows, vocab), jnp.float32),
        mesh=mesh,
        compiler_params=pltpu.CompilerParams(
            needs_layout_passes=False,
            skip_device_barrier=True,
        ),
        scratch_types=[
            pltpu.VMEM((_RG, seq), jnp.int32),
            pltpu.VMEM((_RG, _CW), jnp.float32),
            pltpu.VMEM((_RG, _CW), jnp.float32),
            pltpu.VMEM((_RG, _CW), jnp.float32),
            pltpu.VMEM((_RG, _MT), jnp.float32),
            pltpu.VMEM((16,), jnp.int32),
            pltpu.SemaphoreType.DMA,
            pltpu.SemaphoreType.DMA,
            pltpu.SemaphoreType.DMA,
            pltpu.SemaphoreType.DMA,
            pltpu.SemaphoreType.DMA,
            pltpu.SemaphoreType.DMA,
            pltpu.SemaphoreType.DMA,
            pltpu.SemaphoreType.DMA,
        ],
    )
    return run(valid_arr, tokens, lprobs)


# speedup vs baseline: 2.5572x; 1.0103x over previous
import jax
import jax.numpy as jnp
from jax import lax
from jax.experimental import pallas as pl
from jax.experimental.pallas import tpu as pltpu
from jax.experimental.pallas import tpu_sc as plsc

_N = 3
_RG = 8
_CW = 4608
_NSTRIPE = 4
_MT = 32


def _body_fn(rows, seq, vocab, valid_hbm, tokens_hbm, lprobs_hbm, out_hbm,
             tok_v, bufs, mt_v, vld_v, sem_t, sems_l, sems_s, sem_mt):
    npos = seq - _N + 1
    nmatch = (npos + 15) // 16
    mt_off = (vocab // 128) * 128
    starts = [0, 25088, 50048, 75008]
    ends = [25088, 50048, 75008, 99968]
    stripe_chunks = []
    for c in range(_NSTRIPE):
        ch, off = [], starts[c]
        while off < ends[c]:
            w = min(_CW, ends[c] - off)
            ch.append((off, w))
            off += w
        stripe_chunks.append(ch)

    wid = lax.axis_index("c") * 16 + lax.axis_index("s")
    rg = wid // _NSTRIPE
    cs = wid % _NSTRIPE
    row0 = pl.multiple_of(rg * _RG, _RG)

    pltpu.sync_copy(valid_hbm, vld_v)
    valid = vld_v[pl.ds(0, 16)][0] != 0
    lanes = lax.iota(jnp.int32, 16)
    neg_inf = jnp.full((16,), -jnp.inf, dtype=jnp.float32)

    cp_t = pltpu.make_async_copy(tokens_hbm.at[pl.ds(row0, _RG)], tok_v,
                                 sem_t)
    cp_t.start()

    def load_c(off, buf, sem, w=_CW):
        dst = buf if w == buf.shape[1] else buf.at[:, pl.ds(0, w)]
        return pltpu.make_async_copy(
            lprobs_hbm.at[pl.ds(row0, _RG), pl.ds(off, w)], dst, sem)

    def store_c(off, buf, sem, w=_CW):
        src = buf if w == buf.shape[1] else buf.at[:, pl.ds(0, w)]
        return pltpu.make_async_copy(
            src, out_hbm.at[pl.ds(row0, _RG), pl.ds(off, w)], sem)

    start_col = (jnp.where(cs == 0, starts[0],
                 jnp.where(cs == 1, starts[1],
                 jnp.where(cs == 2, starts[2], starts[3])))
                 ).astype(jnp.int32)
    for b in range(3):
        off_b = pl.multiple_of(start_col + b * _CW, 128)
        load_c(off_b, bufs[b], sems_l[b]).start()

    ld_mt = pltpu.make_async_copy(
        lprobs_hbm.at[pl.ds(row0, _RG), pl.ds(mt_off, _MT)], mt_v, sem_mt)
    st_mt = pltpu.make_async_copy(
        mt_v, out_hbm.at[pl.ds(row0, _RG), pl.ds(mt_off, _MT)], sem_mt)

    @pl.when(cs == _NSTRIPE - 1)
    def _():
        ld_mt.start()

    cp_t.wait()

    def row_match(r, bits):
        rvec = jnp.full((16,), 0, jnp.int32) + r
        l0 = plsc.load_gather(
            tok_v, [rvec, jnp.full((16,), seq - 2, jnp.int32)])[0]
        l1 = plsc.load_gather(
            tok_v, [rvec, jnp.full((16,), seq - 1, jnp.int32)])[0]

        def mbody(i, acc):
            for u in range(4):
                idx = lanes + (i * 4 + u) * 16
                t0 = plsc.load_gather(tok_v, [rvec, idx])
                t1 = plsc.load_gather(tok_v, [rvec, jnp.minimum(idx + 1,
                                                                seq - 1)])
                acc = acc | ((idx < npos) & (t0 == l0) & (t1 == l1))
            return acc

        acc = lax.fori_loop(0, nmatch // 4, mbody,
                            jnp.zeros((16,), jnp.bool_))
        return bits | (jnp.any(acc).astype(jnp.int32) << r)

    anybits = lax.fori_loop(0, _RG, row_match, jnp.int32(0))
    anybits = jnp.where(valid, anybits, 0)

    def scatter(buf, off, w):
        @pl.when(anybits != 0)
        def _():
            def rloop(r, c):
                rvec = jnp.full((16,), 0, jnp.int32) + r

                @pl.when(((anybits >> r) & 1) != 0)
                def _():
                    l0 = plsc.load_gather(
                        tok_v,
                        [rvec, jnp.full((16,), seq - 2, jnp.int32)])[0]
                    l1 = plsc.load_gather(
                        tok_v,
                        [rvec, jnp.full((16,), seq - 1, jnp.int32)])[0]

                    def sbody(i, c2):
                        idx = lanes + i * 16
                        t0 = plsc.load_gather(
                            tok_v, [rvec, jnp.minimum(idx, seq - 1)])
                        t1 = plsc.load_gather(
                            tok_v, [rvec, jnp.minimum(idx + 1, seq - 1)])
                        t2 = plsc.load_gather(
                            tok_v, [rvec, jnp.minimum(idx + 2, seq - 1)])
                        m = ((idx < npos) & (t0 == l0) & (t1 == l1)
                             & (t2 >= off) & (t2 < off + w))
                        plsc.store_scatter(
                            buf, [rvec, jnp.where(m, t2 - off, 0)],
                            neg_inf, mask=m)
                        return c2

                    lax.fori_loop(0, nmatch, sbody, 0)

                return c

            lax.fori_loop(0, _RG, rloop, 0)

    def stripe_pipeline(chunks, has_mt):
        n = len(chunks)
        lds, sts = {}, {}
        for b in range(3):
            off, w = chunks[b]
            lds[b] = load_c(off, bufs[b], sems_l[b], w=w)
        for jj in range(n):
            b = jj % 3
            off, w = chunks[jj]
            lds[jj].wait()
            scatter(bufs[b], off, w)
            st = store_c(off, bufs[b], sems_s[b], w=w)
            st.start()
            sts[jj] = st
            if jj >= 2 and jj + 1 < n:
                sts[jj - 2].wait()
                noff, nw = chunks[jj + 1]
                nld = load_c(noff, bufs[(jj + 1) % 3],
                             sems_l[(jj + 1) % 3], w=nw)
                nld.start()
                lds[jj + 1] = nld
        if has_mt:
            ld_mt.wait()
            scatter(mt_v, mt_off, _MT)
            st_mt.start()
            st_mt.wait()
        for jj in range(max(0, n - 2), n):
            sts[jj].wait()

    branches = []
    for c in range(_NSTRIPE):
        branches.append(lambda ch=stripe_chunks[c], t=(c == _NSTRIPE - 1):
                        stripe_pipeline(ch, t))
    lax.switch(cs, branches)


def kernel(tokens, lprobs, bsz, step, beam_size, no_repeat_ngram_size):
    rows, seq = tokens.shape
    vocab = lprobs.shape[1]
    valid = (
        (rows == bsz * beam_size)
        & (step == seq - 1)
        & (no_repeat_ngram_size == _N)
    )
    valid_arr = jnp.full((16,), 0, dtype=jnp.int32) + valid.astype(jnp.int32)

    mesh = plsc.VectorSubcoreMesh(core_axis_name="c", subcore_axis_name="s")

    def body(valid_hbm, tokens_hbm, lprobs_hbm, out_hbm, tok_v, buf_0,
             buf_1, buf_2, mt_v, vld_v, sem_t, sem_l0, sem_l1, sem_l2,
             sem_s0, sem_s1, sem_s2, sem_mt):
        _body_fn(rows, seq, vocab, valid_hbm, tokens_hbm, lprobs_hbm,
                 out_hbm, tok_v, [buf_0, buf_1, buf_2], mt_v, vld_v,
                 sem_t, [sem_l0, sem_l1, sem_l2],
                 [sem_s0, sem_s1, sem_s2], sem_mt)

    run = pl.kernel(
        body,
        out_type=jax.ShapeDtypeStruct((rows, vocab), jnp.float32),
        mesh=mesh,
        compiler_params=pltpu.CompilerParams(
            needs_layout_passes=False,
            skip_device_barrier=True,
        ),
        scratch_types=[
            pltpu.VMEM((_RG, seq), jnp.int32),
            pltpu.VMEM((_RG, _CW), jnp.float32),
            pltpu.VMEM((_RG, _CW), jnp.float32),
            pltpu.VMEM((_RG, _CW), jnp.float32),
            pltpu.VMEM((_RG, _MT), jnp.float32),
            pltpu.VMEM((16,), jnp.int32),
            pltpu.SemaphoreType.DMA,
            pltpu.SemaphoreType.DMA,
            pltpu.SemaphoreType.DMA,
            pltpu.SemaphoreType.DMA,
            pltpu.SemaphoreType.DMA,
            pltpu.SemaphoreType.DMA,
            pltpu.SemaphoreType.DMA,
            pltpu.SemaphoreType.DMA,
        ],
    )
    return run(valid_arr, tokens, lprobs)


# speedup vs baseline: 2.6814x; 1.0486x over previous
import jax
import jax.numpy as jnp
from jax import lax
from jax.experimental import pallas as pl
from jax.experimental.pallas import tpu as pltpu
from jax.experimental.pallas import tpu_sc as plsc

_N = 3
_RG = 8
_CW = 7040
_NSTRIPE = 4
_MT = 32


def _body_fn(rows, seq, vocab, valid_hbm, tokens_hbm, lprobs_hbm, out_hbm,
             tok_v, bufs, mt_v, vld_v, sem_t, sems_l, sems_s, sem_mt):
    npos = seq - _N + 1
    nmatch = (npos + 15) // 16
    mt_off = (vocab // 128) * 128
    starts = [0, 25088, 50048, 75008]
    ends = [25088, 50048, 75008, 99968]
    stripe_chunks = []
    for c in range(_NSTRIPE):
        ch, off = [], starts[c]
        while off < ends[c]:
            w = min(_CW, ends[c] - off)
            ch.append((off, w))
            off += w
        stripe_chunks.append(ch)

    wid = lax.axis_index("c") * 16 + lax.axis_index("s")
    rg = wid // _NSTRIPE
    cs = wid % _NSTRIPE
    row0 = pl.multiple_of(rg * _RG, _RG)

    pltpu.sync_copy(valid_hbm, vld_v)
    valid = vld_v[pl.ds(0, 16)][0] != 0
    lanes = lax.iota(jnp.int32, 16)
    neg_inf = jnp.full((16,), -jnp.inf, dtype=jnp.float32)

    cp_t = pltpu.make_async_copy(tokens_hbm.at[pl.ds(row0, _RG)], tok_v,
                                 sem_t)
    cp_t.start()

    def load_c(off, buf, sem, w=_CW):
        dst = buf if w == buf.shape[1] else buf.at[:, pl.ds(0, w)]
        return pltpu.make_async_copy(
            lprobs_hbm.at[pl.ds(row0, _RG), pl.ds(off, w)], dst, sem)

    def store_c(off, buf, sem, w=_CW):
        src = buf if w == buf.shape[1] else buf.at[:, pl.ds(0, w)]
        return pltpu.make_async_copy(
            src, out_hbm.at[pl.ds(row0, _RG), pl.ds(off, w)], sem)

    start_col = (jnp.where(cs == 0, starts[0],
                 jnp.where(cs == 1, starts[1],
                 jnp.where(cs == 2, starts[2], starts[3])))
                 ).astype(jnp.int32)
    for b in range(2):
        off_b = pl.multiple_of(start_col + b * _CW, 128)
        load_c(off_b, bufs[b], sems_l[b]).start()

    ld_mt = pltpu.make_async_copy(
        lprobs_hbm.at[pl.ds(row0, _RG), pl.ds(mt_off, _MT)], mt_v, sem_mt)
    st_mt = pltpu.make_async_copy(
        mt_v, out_hbm.at[pl.ds(row0, _RG), pl.ds(mt_off, _MT)], sem_mt)

    @pl.when(cs == _NSTRIPE - 1)
    def _():
        ld_mt.start()

    cp_t.wait()

    def row_match(r, bits):
        rvec = jnp.full((16,), 0, jnp.int32) + r
        l0 = plsc.load_gather(
            tok_v, [rvec, jnp.full((16,), seq - 2, jnp.int32)])[0]
        l1 = plsc.load_gather(
            tok_v, [rvec, jnp.full((16,), seq - 1, jnp.int32)])[0]

        def mbody(i, acc):
            for u in range(4):
                idx = lanes + (i * 4 + u) * 16
                t0 = plsc.load_gather(tok_v, [rvec, idx])
                t1 = plsc.load_gather(tok_v, [rvec, jnp.minimum(idx + 1,
                                                                seq - 1)])
                acc = acc | ((idx < npos) & (t0 == l0) & (t1 == l1))
            return acc

        acc = lax.fori_loop(0, nmatch // 4, mbody,
                            jnp.zeros((16,), jnp.bool_))
        return bits | (jnp.any(acc).astype(jnp.int32) << r)

    anybits = lax.fori_loop(0, _RG, row_match, jnp.int32(0))
    anybits = jnp.where(valid, anybits, 0)

    def scatter(buf, off, w):
        @pl.when(anybits != 0)
        def _():
            def rloop(r, c):
                rvec = jnp.full((16,), 0, jnp.int32) + r

                @pl.when(((anybits >> r) & 1) != 0)
                def _():
                    l0 = plsc.load_gather(
                        tok_v,
                        [rvec, jnp.full((16,), seq - 2, jnp.int32)])[0]
                    l1 = plsc.load_gather(
                        tok_v,
                        [rvec, jnp.full((16,), seq - 1, jnp.int32)])[0]

                    def sbody(i, c2):
                        idx = lanes + i * 16
                        t0 = plsc.load_gather(
                            tok_v, [rvec, jnp.minimum(idx, seq - 1)])
                        t1 = plsc.load_gather(
                            tok_v, [rvec, jnp.minimum(idx + 1, seq - 1)])
                        t2 = plsc.load_gather(
                            tok_v, [rvec, jnp.minimum(idx + 2, seq - 1)])
                        m = ((idx < npos) & (t0 == l0) & (t1 == l1)
                             & (t2 >= off) & (t2 < off + w))
                        plsc.store_scatter(
                            buf, [rvec, jnp.where(m, t2 - off, 0)],
                            neg_inf, mask=m)
                        return c2

                    lax.fori_loop(0, nmatch, sbody, 0)

                return c

            lax.fori_loop(0, _RG, rloop, 0)

    def stripe_pipeline(chunks, has_mt):
        n = len(chunks)
        lds, sts = {}, {}
        for b in range(2):
            off, w = chunks[b]
            lds[b] = load_c(off, bufs[b], sems_l[b], w=w)
        for jj in range(n):
            b = jj % 2
            off, w = chunks[jj]
            lds[jj].wait()
            scatter(bufs[b], off, w)
            st = store_c(off, bufs[b], sems_s[b], w=w)
            st.start()
            sts[jj] = st
            if jj >= 1 and jj + 1 < n:
                sts[jj - 1].wait()
                noff, nw = chunks[jj + 1]
                nld = load_c(noff, bufs[(jj + 1) % 2],
                             sems_l[(jj + 1) % 2], w=nw)
                nld.start()
                lds[jj + 1] = nld
        if has_mt:
            ld_mt.wait()
            scatter(mt_v, mt_off, _MT)
            st_mt.start()
            st_mt.wait()
        for jj in range(max(0, n - 2), n):
            sts[jj].wait()

    branches = []
    for c in range(_NSTRIPE):
        branches.append(lambda ch=stripe_chunks[c], t=(c == _NSTRIPE - 1):
                        stripe_pipeline(ch, t))
    lax.switch(cs, branches)


def kernel(tokens, lprobs, bsz, step, beam_size, no_repeat_ngram_size):
    rows, seq = tokens.shape
    vocab = lprobs.shape[1]
    valid = (
        (rows == bsz * beam_size)
        & (step == seq - 1)
        & (no_repeat_ngram_size == _N)
    )
    valid_arr = jnp.full((16,), 0, dtype=jnp.int32) + valid.astype(jnp.int32)

    mesh = plsc.VectorSubcoreMesh(core_axis_name="c", subcore_axis_name="s")

    def body(valid_hbm, tokens_hbm, lprobs_hbm, out_hbm, tok_v, buf_0,
             buf_1, mt_v, vld_v, sem_t, sem_l0, sem_l1,
             sem_s0, sem_s1, sem_mt):
        _body_fn(rows, seq, vocab, valid_hbm, tokens_hbm, lprobs_hbm,
                 out_hbm, tok_v, [buf_0, buf_1], mt_v, vld_v,
                 sem_t, [sem_l0, sem_l1],
                 [sem_s0, sem_s1], sem_mt)

    run = pl.kernel(
        body,
        out_type=jax.ShapeDtypeStruct((rows, vocab), jnp.float32),
        mesh=mesh,
        compiler_params=pltpu.CompilerParams(
            needs_layout_passes=False,
            skip_device_barrier=True,
        ),
        scratch_types=[
            pltpu.VMEM((_RG, seq), jnp.int32),
            pltpu.VMEM((_RG, _CW), jnp.float32),
            pltpu.VMEM((_RG, _CW), jnp.float32),
            pltpu.VMEM((_RG, _MT), jnp.float32),
            pltpu.VMEM((16,), jnp.int32),
            pltpu.SemaphoreType.DMA,
            pltpu.SemaphoreType.DMA,
            pltpu.SemaphoreType.DMA,
            pltpu.SemaphoreType.DMA,
            pltpu.SemaphoreType.DMA,
            pltpu.SemaphoreType.DMA,
        ],
    )
    return run(valid_arr, tokens, lprobs)


# speedup vs baseline: 2.7832x; 1.0380x over previous
import jax
import jax.numpy as jnp
from jax import lax
from jax.experimental import pallas as pl
from jax.experimental.pallas import tpu as pltpu
from jax.experimental.pallas import tpu_sc as plsc

_N = 3
_RG = 8
_CW = 7040
_NSTRIPE = 4
_MT = 32


def _body_fn(rows, seq, vocab, valid_hbm, tokens_hbm, lprobs_hbm, out_hbm,
             tok_v, bufs, mt_v, vld_v, sem_t, sems_l, sems_s, sem_mt):
    npos = seq - _N + 1
    nmatch = (npos + 15) // 16
    mt_off = (vocab // 128) * 128
    starts = [0, 25088, 50048, 75008]
    ends = [25088, 50048, 75008, 99968]
    stripe_chunks = []
    for c in range(_NSTRIPE):
        ch, off = [], starts[c]
        while off < ends[c]:
            w = min(_CW, ends[c] - off)
            ch.append((off, w))
            off += w
        stripe_chunks.append(ch)

    wid = lax.axis_index("c") * 16 + lax.axis_index("s")
    rg = wid // _NSTRIPE
    cs = wid % _NSTRIPE
    row0 = pl.multiple_of(rg * _RG, _RG)

    pltpu.sync_copy(valid_hbm, vld_v)
    valid = vld_v[pl.ds(0, 16)][0] != 0
    lanes = lax.iota(jnp.int32, 16)
    neg_inf = jnp.full((16,), -jnp.inf, dtype=jnp.float32)

    cp_t = pltpu.make_async_copy(tokens_hbm.at[pl.ds(row0, _RG)], tok_v,
                                 sem_t)
    cp_t.start()

    def load_c(off, buf, sem, w=_CW):
        dst = buf if w == buf.shape[1] else buf.at[:, pl.ds(0, w)]
        return pltpu.make_async_copy(
            lprobs_hbm.at[pl.ds(row0, _RG), pl.ds(off, w)], dst, sem)

    def store_c(off, buf, sem, w=_CW):
        src = buf if w == buf.shape[1] else buf.at[:, pl.ds(0, w)]
        return pltpu.make_async_copy(
            src, out_hbm.at[pl.ds(row0, _RG), pl.ds(off, w)], sem)

    start_col = (jnp.where(cs == 0, starts[0],
                 jnp.where(cs == 1, starts[1],
                 jnp.where(cs == 2, starts[2], starts[3])))
                 ).astype(jnp.int32)
    end_col = (jnp.where(cs == 0, ends[0],
               jnp.where(cs == 1, ends[1],
               jnp.where(cs == 2, ends[2], vocab)))
               ).astype(jnp.int32)
    for b in range(2):
        off_b = pl.multiple_of(start_col + b * _CW, 128)
        load_c(off_b, bufs[b], sems_l[b]).start()

    ld_mt = pltpu.make_async_copy(
        lprobs_hbm.at[pl.ds(row0, _RG), pl.ds(mt_off, _MT)], mt_v, sem_mt)
    st_mt = pltpu.make_async_copy(
        mt_v, out_hbm.at[pl.ds(row0, _RG), pl.ds(mt_off, _MT)], sem_mt)

    @pl.when(cs == _NSTRIPE - 1)
    def _():
        ld_mt.start()

    cp_t.wait()

    def row_match(r, bits):
        rvec = jnp.full((16,), 0, jnp.int32) + r
        l0 = plsc.load_gather(
            tok_v, [rvec, jnp.full((16,), seq - 2, jnp.int32)])[0]
        l1 = plsc.load_gather(
            tok_v, [rvec, jnp.full((16,), seq - 1, jnp.int32)])[0]

        def mbody(i, acc):
            for u in range(4):
                idx = lanes + (i * 4 + u) * 16
                t0 = plsc.load_gather(tok_v, [rvec, idx])
                t1 = plsc.load_gather(tok_v, [rvec, jnp.minimum(idx + 1,
                                                                seq - 1)])
                acc = acc | ((idx < npos) & (t0 == l0) & (t1 == l1))
            return acc

        acc = lax.fori_loop(0, nmatch // 4, mbody,
                            jnp.zeros((16,), jnp.bool_))
        return bits | (jnp.any(acc).astype(jnp.int32) << r)

    anybits = lax.fori_loop(0, _RG, row_match, jnp.int32(0))
    anybits = jnp.where(valid, anybits, 0)

    def stripe_pipeline(chunks, has_mt):
        n = len(chunks)
        lds, sts = {}, {}
        for b in range(2):
            off, w = chunks[b]
            lds[b] = load_c(off, bufs[b], sems_l[b], w=w)
        for jj in range(n):
            b = jj % 2
            off, w = chunks[jj]
            lds[jj].wait()
            st = store_c(off, bufs[b], sems_s[b], w=w)
            st.start()
            sts[jj] = st
            if jj >= 1 and jj + 1 < n:
                sts[jj - 1].wait()
                noff, nw = chunks[jj + 1]
                nld = load_c(noff, bufs[(jj + 1) % 2],
                             sems_l[(jj + 1) % 2], w=nw)
                nld.start()
                lds[jj + 1] = nld
        if has_mt:
            ld_mt.wait()
            st_mt.start()
            st_mt.wait()
        for jj in range(max(0, n - 2), n):
            sts[jj].wait()

    branches = []
    for c in range(_NSTRIPE):
        branches.append(lambda ch=stripe_chunks[c], t=(c == _NSTRIPE - 1):
                        stripe_pipeline(ch, t))
    lax.switch(cs, branches)

    @pl.when(anybits != 0)
    def _():
        def rloop(r, cr):
            rvec = jnp.full((16,), 0, jnp.int32) + r

            @pl.when(((anybits >> r) & 1) != 0)
            def _():
                l0 = plsc.load_gather(
                    tok_v, [rvec, jnp.full((16,), seq - 2, jnp.int32)])[0]
                l1 = plsc.load_gather(
                    tok_v, [rvec, jnp.full((16,), seq - 1, jnp.int32)])[0]

                def sbody(i, c2):
                    idx = lanes + i * 16
                    t0 = plsc.load_gather(tok_v, [rvec, idx])
                    t1 = plsc.load_gather(
                        tok_v, [rvec, jnp.minimum(idx + 1, seq - 1)])
                    t2 = plsc.load_gather(
                        tok_v, [rvec, jnp.minimum(idx + 2, seq - 1)])
                    m = ((idx < npos) & (t0 == l0) & (t1 == l1)
                         & (t2 >= start_col) & (t2 < end_col))

                    mi = m.astype(jnp.int32)

                    @pl.when(jnp.any(m))
                    def _():
                        for l in range(16):
                            b = t2[l]

                            @pl.when(mi[l] != 0)
                            def _(b=b, r=r):
                                is_mt = b >= mt_off

                                @pl.when(~is_mt)
                                def _():
                                    colt = pl.multiple_of(
                                        (b >> 7) << 7, 128)
                                    tile = bufs[0].at[:, pl.ds(0, 128)]
                                    cp = pltpu.make_async_copy(
                                        out_hbm.at[pl.ds(row0, _RG),
                                                   pl.ds(colt, 128)],
                                        tile, sem_mt)
                                    cp.start()
                                    cp.wait()
                                    plsc.store_scatter(
                                        tile,
                                        [jnp.full((16,), 0, jnp.int32) + r,
                                         jnp.full((16,), 0, jnp.int32)
                                         + (b - colt)],
                                        neg_inf, mask=lanes == 0)
                                    cp2 = pltpu.make_async_copy(
                                        tile,
                                        out_hbm.at[pl.ds(row0, _RG),
                                                   pl.ds(colt, 128)],
                                        sem_mt)
                                    cp2.start()
                                    cp2.wait()

                                @pl.when(is_mt)
                                def _():
                                    cp = pltpu.make_async_copy(
                                        out_hbm.at[pl.ds(row0, _RG),
                                                   pl.ds(mt_off, _MT)],
                                        mt_v, sem_mt)
                                    cp.start()
                                    cp.wait()
                                    plsc.store_scatter(
                                        mt_v,
                                        [jnp.full((16,), 0, jnp.int32) + r,
                                         jnp.full((16,), 0, jnp.int32)
                                         + (b - mt_off)],
                                        neg_inf, mask=lanes == 0)
                                    cp2 = pltpu.make_async_copy(
                                        mt_v,
                                        out_hbm.at[pl.ds(row0, _RG),
                                                   pl.ds(mt_off, _MT)],
                                        sem_mt)
                                    cp2.start()
                                    cp2.wait()

                    return c2

                lax.fori_loop(0, nmatch, sbody, 0)

            return cr

        lax.fori_loop(0, _RG, rloop, 0)


def kernel(tokens, lprobs, bsz, step, beam_size, no_repeat_ngram_size):
    rows, seq = tokens.shape
    vocab = lprobs.shape[1]
    valid = (
        (rows == bsz * beam_size)
        & (step == seq - 1)
        & (no_repeat_ngram_size == _N)
    )
    valid_arr = jnp.full((16,), 0, dtype=jnp.int32) + valid.astype(jnp.int32)

    mesh = plsc.VectorSubcoreMesh(core_axis_name="c", subcore_axis_name="s")

    def body(valid_hbm, tokens_hbm, lprobs_hbm, out_hbm, tok_v, buf_0,
             buf_1, mt_v, vld_v, sem_t, sem_l0, sem_l1,
             sem_s0, sem_s1, sem_mt):
        _body_fn(rows, seq, vocab, valid_hbm, tokens_hbm, lprobs_hbm,
                 out_hbm, tok_v, [buf_0, buf_1], mt_v, vld_v,
                 sem_t, [sem_l0, sem_l1],
                 [sem_s0, sem_s1], sem_mt)

    run = pl.kernel(
        body,
        out_type=jax.ShapeDtypeStruct((rows, vocab), jnp.float32),
        mesh=mesh,
        compiler_params=pltpu.CompilerParams(
            needs_layout_passes=False,
            skip_device_barrier=True,
        ),
        scratch_types=[
            pltpu.VMEM((_RG, seq), jnp.int32),
            pltpu.VMEM((_RG, _CW), jnp.float32),
            pltpu.VMEM((_RG, _CW), jnp.float32),
            pltpu.VMEM((_RG, _MT), jnp.float32),
            pltpu.VMEM((16,), jnp.int32),
            pltpu.SemaphoreType.DMA,
            pltpu.SemaphoreType.DMA,
            pltpu.SemaphoreType.DMA,
            pltpu.SemaphoreType.DMA,
            pltpu.SemaphoreType.DMA,
            pltpu.SemaphoreType.DMA,
        ],
    )
    return run(valid_arr, tokens, lprobs)


# speedup vs baseline: 2.8401x; 1.0204x over previous
import jax
import jax.numpy as jnp
from jax import lax
from jax.experimental import pallas as pl
from jax.experimental.pallas import tpu as pltpu
from jax.experimental.pallas import tpu_sc as plsc

_N = 3
_RG = 8
_CW = 14080
_NSTRIPE = 4
_MT = 32


def _body_fn(rows, seq, vocab, valid_hbm, tokens_hbm, lprobs_hbm, out_hbm,
             tok_v, bufs, mt_v, vld_v, sem_t, sems_l, sems_s, sem_mt):
    npos = seq - _N + 1
    nmatch = (npos + 15) // 16
    mt_off = (vocab // 128) * 128
    starts = [0, 25088, 50048, 75008]
    w2 = [25088 - _CW, 24960 - _CW]

    wid = lax.axis_index("c") * 16 + lax.axis_index("s")
    rg = wid // _NSTRIPE
    cs = wid % _NSTRIPE
    row0 = pl.multiple_of(rg * _RG, _RG)

    pltpu.sync_copy(valid_hbm, vld_v)
    valid = vld_v[pl.ds(0, 16)][0] != 0
    lanes = lax.iota(jnp.int32, 16)
    neg_inf = jnp.full((16,), -jnp.inf, dtype=jnp.float32)

    cp_t = pltpu.make_async_copy(tokens_hbm.at[pl.ds(row0, _RG)], tok_v,
                                 sem_t)
    cp_t.start()

    def load_c(off, buf, sem, w=_CW):
        dst = buf if w == buf.shape[1] else buf.at[:, pl.ds(0, w)]
        return pltpu.make_async_copy(
            lprobs_hbm.at[pl.ds(row0, _RG), pl.ds(off, w)], dst, sem)

    def store_c(off, buf, sem, w=_CW):
        src = buf if w == buf.shape[1] else buf.at[:, pl.ds(0, w)]
        return pltpu.make_async_copy(
            src, out_hbm.at[pl.ds(row0, _RG), pl.ds(off, w)], sem)

    start_col = (jnp.where(cs == 0, starts[0],
                 jnp.where(cs == 1, starts[1],
                 jnp.where(cs == 2, starts[2], starts[3])))
                 ).astype(jnp.int32)
    end_col = (jnp.where(cs == 0, 25088,
               jnp.where(cs == 1, 50048,
               jnp.where(cs == 2, 75008, vocab)))
               ).astype(jnp.int32)
    off_c0 = pl.multiple_of(start_col, 128)
    load_c(off_c0, bufs[0], sems_l[0]).start()

    ld_mt = pltpu.make_async_copy(
        lprobs_hbm.at[pl.ds(row0, _RG), pl.ds(mt_off, _MT)], mt_v, sem_mt)
    st_mt = pltpu.make_async_copy(
        mt_v, out_hbm.at[pl.ds(row0, _RG), pl.ds(mt_off, _MT)], sem_mt)

    @pl.when(cs == _NSTRIPE - 1)
    def _():
        ld_mt.start()

    cp_t.wait()

    def row_match(r, bits):
        rvec = jnp.full((16,), 0, jnp.int32) + r
        l0 = plsc.load_gather(
            tok_v, [rvec, jnp.full((16,), seq - 2, jnp.int32)])[0]
        l1 = plsc.load_gather(
            tok_v, [rvec, jnp.full((16,), seq - 1, jnp.int32)])[0]

        def mbody(i, acc):
            for u in range(4):
                idx = lanes + (i * 4 + u) * 16
                t0 = plsc.load_gather(tok_v, [rvec, idx])
                t1 = plsc.load_gather(tok_v, [rvec, jnp.minimum(idx + 1,
                                                                seq - 1)])
                acc = acc | ((idx < npos) & (t0 == l0) & (t1 == l1))
            return acc

        acc = lax.fori_loop(0, nmatch // 4, mbody,
                            jnp.zeros((16,), jnp.bool_))
        return bits | (jnp.any(acc).astype(jnp.int32) << r)

    anybits = lax.fori_loop(0, _RG, row_match, jnp.int32(0))
    anybits = jnp.where(valid, anybits, 0)

    ld0 = load_c(off_c0, bufs[0], sems_l[0])
    ld0.wait()
    st0 = store_c(off_c0, bufs[0], sems_s[0])
    st0.start()
    st0.wait()
    off_c1 = pl.multiple_of(start_col + _CW, 128)
    ld1_a = load_c(off_c1, bufs[0], sems_l[0], w=w2[0])
    ld1_b = load_c(off_c1, bufs[0], sems_l[0], w=w2[1])

    @pl.when(cs == 0)
    def _():
        ld1_a.start()

    @pl.when(cs != 0)
    def _():
        ld1_b.start()

    @pl.when(cs == _NSTRIPE - 1)
    def _():
        ld_mt.wait()
        st_mt.start()
        st_mt.wait()

    @pl.when(cs == 0)
    def _():
        ld1_a.wait()
        st = store_c(off_c1, bufs[0], sems_s[0], w=w2[0])
        st.start()
        st.wait()

    @pl.when(cs != 0)
    def _():
        ld1_b.wait()
        st = store_c(off_c1, bufs[0], sems_s[0], w=w2[1])
        st.start()
        st.wait()

    @pl.when(anybits != 0)
    def _():
        def rloop(r, cr):
            rvec = jnp.full((16,), 0, jnp.int32) + r

            @pl.when(((anybits >> r) & 1) != 0)
            def _():
                l0 = plsc.load_gather(
                    tok_v, [rvec, jnp.full((16,), seq - 2, jnp.int32)])[0]
                l1 = plsc.load_gather(
                    tok_v, [rvec, jnp.full((16,), seq - 1, jnp.int32)])[0]

                def sbody(i, c2):
                    idx = lanes + i * 16
                    t0 = plsc.load_gather(tok_v, [rvec, idx])
                    t1 = plsc.load_gather(
                        tok_v, [rvec, jnp.minimum(idx + 1, seq - 1)])
                    t2 = plsc.load_gather(
                        tok_v, [rvec, jnp.minimum(idx + 2, seq - 1)])
                    m = ((idx < npos) & (t0 == l0) & (t1 == l1)
                         & (t2 >= start_col) & (t2 < end_col))

                    mi = m.astype(jnp.int32)

                    @pl.when(jnp.any(m))
                    def _():
                        for l in range(16):
                            b = t2[l]

                            @pl.when(mi[l] != 0)
                            def _(b=b, r=r):
                                is_mt = b >= mt_off

                                @pl.when(~is_mt)
                                def _():
                                    colt = pl.multiple_of(
                                        (b >> 7) << 7, 128)
                                    tile = bufs[0].at[:, pl.ds(0, 128)]
                                    cp = pltpu.make_async_copy(
                                        out_hbm.at[pl.ds(row0, _RG),
                                                   pl.ds(colt, 128)],
                                        tile, sem_mt)
                                    cp.start()
                                    cp.wait()
                                    plsc.store_scatter(
                                        tile,
                                        [jnp.full((16,), 0, jnp.int32) + r,
                                         jnp.full((16,), 0, jnp.int32)
                                         + (b - colt)],
                                        neg_inf, mask=lanes == 0)
                                    cp2 = pltpu.make_async_copy(
                                        tile,
                                        out_hbm.at[pl.ds(row0, _RG),
                                                   pl.ds(colt, 128)],
                                        sem_mt)
                                    cp2.start()
                                    cp2.wait()

                                @pl.when(is_mt)
                                def _():
                                    cp = pltpu.make_async_copy(
                                        out_hbm.at[pl.ds(row0, _RG),
                                                   pl.ds(mt_off, _MT)],
                                        mt_v, sem_mt)
                                    cp.start()
                                    cp.wait()
                                    plsc.store_scatter(
                                        mt_v,
                                        [jnp.full((16,), 0, jnp.int32) + r,
                                         jnp.full((16,), 0, jnp.int32)
                                         + (b - mt_off)],
                                        neg_inf, mask=lanes == 0)
                                    cp2 = pltpu.make_async_copy(
                                        mt_v,
                                        out_hbm.at[pl.ds(row0, _RG),
                                                   pl.ds(mt_off, _MT)],
                                        sem_mt)
                                    cp2.start()
                                    cp2.wait()

                    return c2

                lax.fori_loop(0, nmatch, sbody, 0)

            return cr

        lax.fori_loop(0, _RG, rloop, 0)


def kernel(tokens, lprobs, bsz, step, beam_size, no_repeat_ngram_size):
    rows, seq = tokens.shape
    vocab = lprobs.shape[1]
    valid = (
        (rows == bsz * beam_size)
        & (step == seq - 1)
        & (no_repeat_ngram_size == _N)
    )
    valid_arr = jnp.full((16,), 0, dtype=jnp.int32) + valid.astype(jnp.int32)

    mesh = plsc.VectorSubcoreMesh(core_axis_name="c", subcore_axis_name="s")

    def body(valid_hbm, tokens_hbm, lprobs_hbm, out_hbm, tok_v, buf_0,
             mt_v, vld_v, sem_t, sem_l0, sem_s0, sem_mt):
        _body_fn(rows, seq, vocab, valid_hbm, tokens_hbm, lprobs_hbm,
                 out_hbm, tok_v, [buf_0], mt_v, vld_v,
                 sem_t, [sem_l0], [sem_s0], sem_mt)

    run = pl.kernel(
        body,
        out_type=jax.ShapeDtypeStruct((rows, vocab), jnp.float32),
        mesh=mesh,
        compiler_params=pltpu.CompilerParams(
            needs_layout_passes=False,
            skip_device_barrier=True,
        ),
        scratch_types=[
            pltpu.VMEM((_RG, seq), jnp.int32),
            pltpu.VMEM((_RG, _CW), jnp.float32),
            pltpu.VMEM((_RG, _MT), jnp.float32),
            pltpu.VMEM((16,), jnp.int32),
            pltpu.SemaphoreType.DMA,
            pltpu.SemaphoreType.DMA,
            pltpu.SemaphoreType.DMA,
            pltpu.SemaphoreType.DMA,
        ],
    )
    return run(valid_arr, tokens, lprobs)


# speedup vs baseline: 2.9255x; 1.0301x over previous
import jax
import jax.numpy as jnp
from jax import lax
from jax.experimental import pallas as pl
from jax.experimental.pallas import tpu as pltpu
from jax.experimental.pallas import tpu_sc as plsc

_N = 3
_RG = 8
_CW = 14080
_NSTRIPE = 4
_MT = 32


def _body_fn(rows, seq, vocab, valid_hbm, tokens_hbm, lprobs_hbm, out_hbm,
             tok_v, bufs, mt_v, vld_v, sem_t, sems_l, sems_s, sem_mt):
    npos = seq - _N + 1
    nmatch = (npos + 15) // 16
    mt_off = (vocab // 128) * 128
    starts = [0, 25088, 50048, 75008]
    w2 = [25088 - _CW, 24960 - _CW]

    wid = lax.axis_index("c") * 16 + lax.axis_index("s")
    rg = wid // _NSTRIPE
    cs = wid % _NSTRIPE
    row0 = pl.multiple_of(rg * _RG, _RG)

    lanes = lax.iota(jnp.int32, 16)
    neg_inf = jnp.full((16,), -jnp.inf, dtype=jnp.float32)

    cp_t = pltpu.make_async_copy(tokens_hbm.at[pl.ds(row0, _RG)], tok_v,
                                 sem_t)
    cp_t.start()

    def load_c(off, buf, sem, w=_CW):
        dst = buf if w == buf.shape[1] else buf.at[:, pl.ds(0, w)]
        return pltpu.make_async_copy(
            lprobs_hbm.at[pl.ds(row0, _RG), pl.ds(off, w)], dst, sem)

    def store_c(off, buf, sem, w=_CW):
        src = buf if w == buf.shape[1] else buf.at[:, pl.ds(0, w)]
        return pltpu.make_async_copy(
            src, out_hbm.at[pl.ds(row0, _RG), pl.ds(off, w)], sem)

    start_col = (jnp.where(cs == 0, starts[0],
                 jnp.where(cs == 1, starts[1],
                 jnp.where(cs == 2, starts[2], starts[3])))
                 ).astype(jnp.int32)
    end_col = (jnp.where(cs == 0, 25088,
               jnp.where(cs == 1, 50048,
               jnp.where(cs == 2, 75008, vocab)))
               ).astype(jnp.int32)
    off_c0 = pl.multiple_of(start_col, 128)
    load_c(off_c0, bufs[0], sems_l[0]).start()

    ld_mt = pltpu.make_async_copy(
        lprobs_hbm.at[pl.ds(row0, _RG), pl.ds(mt_off, _MT)], mt_v, sem_mt)
    st_mt = pltpu.make_async_copy(
        mt_v, out_hbm.at[pl.ds(row0, _RG), pl.ds(mt_off, _MT)], sem_mt)

    @pl.when(cs == _NSTRIPE - 1)
    def _():
        ld_mt.start()

    pltpu.sync_copy(valid_hbm, vld_v)
    valid = vld_v[pl.ds(0, 16)][0] != 0
    cp_t.wait()

    def row_match(r, bits):
        rvec = jnp.full((16,), 0, jnp.int32) + r
        l0 = plsc.load_gather(
            tok_v, [rvec, jnp.full((16,), seq - 2, jnp.int32)])[0]
        l1 = plsc.load_gather(
            tok_v, [rvec, jnp.full((16,), seq - 1, jnp.int32)])[0]

        def mbody(i, acc):
            for u in range(4):
                idx = lanes + (i * 4 + u) * 16
                t0 = plsc.load_gather(tok_v, [rvec, idx])
                t1 = plsc.load_gather(tok_v, [rvec, jnp.minimum(idx + 1,
                                                                seq - 1)])
                acc = acc | ((idx < npos) & (t0 == l0) & (t1 == l1))
            return acc

        acc = lax.fori_loop(0, nmatch // 4, mbody,
                            jnp.zeros((16,), jnp.bool_))
        return bits | (jnp.any(acc).astype(jnp.int32) << r)

    anybits = lax.fori_loop(0, _RG, row_match, jnp.int32(0))
    anybits = jnp.where(valid, anybits, 0)

    ld0 = load_c(off_c0, bufs[0], sems_l[0])
    ld0.wait()
    st0 = store_c(off_c0, bufs[0], sems_s[0])
    st0.start()
    st0.wait()
    off_c1 = pl.multiple_of(start_col + _CW, 128)
    ld1_a = load_c(off_c1, bufs[0], sems_l[0], w=w2[0])
    ld1_b = load_c(off_c1, bufs[0], sems_l[0], w=w2[1])

    @pl.when(cs == 0)
    def _():
        ld1_a.start()

    @pl.when(cs != 0)
    def _():
        ld1_b.start()

    @pl.when(cs == _NSTRIPE - 1)
    def _():
        ld_mt.wait()
        st_mt.start()
        st_mt.wait()

    @pl.when(cs == 0)
    def _():
        ld1_a.wait()
        st = store_c(off_c1, bufs[0], sems_s[0], w=w2[0])
        st.start()
        st.wait()

    @pl.when(cs != 0)
    def _():
        ld1_b.wait()
        st = store_c(off_c1, bufs[0], sems_s[0], w=w2[1])
        st.start()
        st.wait()

    @pl.when(anybits != 0)
    def _():
        def rloop(r, cr):
            rvec = jnp.full((16,), 0, jnp.int32) + r

            @pl.when(((anybits >> r) & 1) != 0)
            def _():
                l0 = plsc.load_gather(
                    tok_v, [rvec, jnp.full((16,), seq - 2, jnp.int32)])[0]
                l1 = plsc.load_gather(
                    tok_v, [rvec, jnp.full((16,), seq - 1, jnp.int32)])[0]

                def sbody(i, c2):
                    idx = lanes + i * 16
                    t0 = plsc.load_gather(tok_v, [rvec, idx])
                    t1 = plsc.load_gather(
                        tok_v, [rvec, jnp.minimum(idx + 1, seq - 1)])
                    t2 = plsc.load_gather(
                        tok_v, [rvec, jnp.minimum(idx + 2, seq - 1)])
                    m = ((idx < npos) & (t0 == l0) & (t1 == l1)
                         & (t2 >= start_col) & (t2 < end_col))

                    mi = m.astype(jnp.int32)

                    @pl.when(jnp.any(m))
                    def _():
                        for l in range(16):
                            b = t2[l]

                            @pl.when(mi[l] != 0)
                            def _(b=b, r=r):
                                is_mt = b >= mt_off

                                @pl.when(~is_mt)
                                def _():
                                    colt = pl.multiple_of(
                                        (b >> 7) << 7, 128)
                                    tile = bufs[0].at[:, pl.ds(0, 128)]
                                    cp = pltpu.make_async_copy(
                                        out_hbm.at[pl.ds(row0, _RG),
                                                   pl.ds(colt, 128)],
                                        tile, sem_mt)
                                    cp.start()
                                    cp.wait()
                                    plsc.store_scatter(
                                        tile,
                                        [jnp.full((16,), 0, jnp.int32) + r,
                                         jnp.full((16,), 0, jnp.int32)
                                         + (b - colt)],
                                        neg_inf, mask=lanes == 0)
                                    cp2 = pltpu.make_async_copy(
                                        tile,
                                        out_hbm.at[pl.ds(row0, _RG),
                                                   pl.ds(colt, 128)],
                                        sem_mt)
                                    cp2.start()
                                    cp2.wait()

                                @pl.when(is_mt)
                                def _():
                                    cp = pltpu.make_async_copy(
                                        out_hbm.at[pl.ds(row0, _RG),
                                                   pl.ds(mt_off, _MT)],
                                        mt_v, sem_mt)
                                    cp.start()
                                    cp.wait()
                                    plsc.store_scatter(
                                        mt_v,
                                        [jnp.full((16,), 0, jnp.int32) + r,
                                         jnp.full((16,), 0, jnp.int32)
                                         + (b - mt_off)],
                                        neg_inf, mask=lanes == 0)
                                    cp2 = pltpu.make_async_copy(
                                        mt_v,
                                        out_hbm.at[pl.ds(row0, _RG),
                                                   pl.ds(mt_off, _MT)],
                                        sem_mt)
                                    cp2.start()
                                    cp2.wait()

                    return c2

                lax.fori_loop(0, nmatch, sbody, 0)

            return cr

        lax.fori_loop(0, _RG, rloop, 0)


def kernel(tokens, lprobs, bsz, step, beam_size, no_repeat_ngram_size):
    rows, seq = tokens.shape
    vocab = lprobs.shape[1]
    valid = (
        (rows == bsz * beam_size)
        & (step == seq - 1)
        & (no_repeat_ngram_size == _N)
    )
    valid_arr = jnp.full((16,), 0, dtype=jnp.int32) + valid.astype(jnp.int32)

    mesh = plsc.VectorSubcoreMesh(core_axis_name="c", subcore_axis_name="s")

    def body(valid_hbm, tokens_hbm, lprobs_hbm, out_hbm, tok_v, buf_0,
             mt_v, vld_v, sem_t, sem_l0, sem_s0, sem_mt):
        _body_fn(rows, seq, vocab, valid_hbm, tokens_hbm, lprobs_hbm,
                 out_hbm, tok_v, [buf_0], mt_v, vld_v,
                 sem_t, [sem_l0], [sem_s0], sem_mt)

    run = pl.kernel(
        body,
        out_type=jax.ShapeDtypeStruct((rows, vocab), jnp.float32),
        mesh=mesh,
        compiler_params=pltpu.CompilerParams(
            needs_layout_passes=False,
            skip_device_barrier=True,
        ),
        scratch_types=[
            pltpu.VMEM((_RG, seq), jnp.int32),
            pltpu.VMEM((_RG, _CW), jnp.float32),
            pltpu.VMEM((_RG, _MT), jnp.float32),
            pltpu.VMEM((16,), jnp.int32),
            pltpu.SemaphoreType.DMA,
            pltpu.SemaphoreType.DMA,
            pltpu.SemaphoreType.DMA,
            pltpu.SemaphoreType.DMA,
        ],
    )
    return run(valid_arr, tokens, lprobs)


# speedup vs baseline: 2.9315x; 1.0020x over previous
import jax
import jax.numpy as jnp
from jax import lax
from jax.experimental import pallas as pl
from jax.experimental.pallas import tpu as pltpu
from jax.experimental.pallas import tpu_sc as plsc

_N = 3
_RG = 8
_CW = 14080
_NSTRIPE = 4
_MT = 32


def _body_fn(rows, seq, vocab, valid_hbm, tokens_hbm, lprobs_hbm, out_hbm,
             tok_v, bufs, mt_v, vld_v, sem_t, sems_l, sems_s, sem_mt):
    npos = seq - _N + 1
    nmatch = (npos + 15) // 16
    mt_off = (vocab // 128) * 128
    starts = [0, 25088, 50048, 75008]
    w2 = [25088 - _CW, 24960 - _CW]

    wid = lax.axis_index("c") * 16 + lax.axis_index("s")
    rg = wid // _NSTRIPE
    cs = wid % _NSTRIPE
    row0 = pl.multiple_of(rg * _RG, _RG)

    lanes = lax.iota(jnp.int32, 16)
    neg_inf = jnp.full((16,), -jnp.inf, dtype=jnp.float32)

    cp_t = pltpu.make_async_copy(tokens_hbm.at[pl.ds(row0, _RG)], tok_v,
                                 sem_t)
    cp_t.start()

    def load_c(off, buf, sem, w=_CW):
        dst = buf if w == buf.shape[1] else buf.at[:, pl.ds(0, w)]
        return pltpu.make_async_copy(
            lprobs_hbm.at[pl.ds(row0, _RG), pl.ds(off, w)], dst, sem)

    def store_c(off, buf, sem, w=_CW):
        src = buf if w == buf.shape[1] else buf.at[:, pl.ds(0, w)]
        return pltpu.make_async_copy(
            src, out_hbm.at[pl.ds(row0, _RG), pl.ds(off, w)], sem)

    start_col = (jnp.where(cs == 0, starts[0],
                 jnp.where(cs == 1, starts[1],
                 jnp.where(cs == 2, starts[2], starts[3])))
                 ).astype(jnp.int32)
    end_col = (jnp.where(cs == 0, 25088,
               jnp.where(cs == 1, 50048,
               jnp.where(cs == 2, 75008, 99968)))
               ).astype(jnp.int32)
    off_c0 = pl.multiple_of(start_col, 128)
    load_c(off_c0, bufs[0], sems_l[0]).start()

    ld_mt = pltpu.make_async_copy(
        lprobs_hbm.at[pl.ds(row0, _RG), pl.ds(mt_off, _MT)], mt_v, sem_mt)
    st_mt = pltpu.make_async_copy(
        mt_v, out_hbm.at[pl.ds(row0, _RG), pl.ds(mt_off, _MT)], sem_mt)

    @pl.when(cs == 1)
    def _():
        ld_mt.start()

    pltpu.sync_copy(valid_hbm, vld_v)
    valid = vld_v[pl.ds(0, 16)][0] != 0
    cp_t.wait()

    def row_match(r, bits):
        rvec = jnp.full((16,), 0, jnp.int32) + r
        l0 = plsc.load_gather(
            tok_v, [rvec, jnp.full((16,), seq - 2, jnp.int32)])[0]
        l1 = plsc.load_gather(
            tok_v, [rvec, jnp.full((16,), seq - 1, jnp.int32)])[0]

        def mbody(i, acc):
            for u in range(4):
                idx = lanes + (i * 4 + u) * 16
                t0 = plsc.load_gather(tok_v, [rvec, idx])
                t1 = plsc.load_gather(tok_v, [rvec, jnp.minimum(idx + 1,
                                                                seq - 1)])
                acc = acc | ((idx < npos) & (t0 == l0) & (t1 == l1))
            return acc

        acc = lax.fori_loop(0, nmatch // 4, mbody,
                            jnp.zeros((16,), jnp.bool_))
        return bits | (jnp.any(acc).astype(jnp.int32) << r)

    anybits = lax.fori_loop(0, _RG, row_match, jnp.int32(0))
    anybits = jnp.where(valid, anybits, 0)

    ld0 = load_c(off_c0, bufs[0], sems_l[0])
    ld0.wait()
    st0 = store_c(off_c0, bufs[0], sems_s[0])
    st0.start()
    st0.wait()
    off_c1 = pl.multiple_of(start_col + _CW, 128)
    ld1_a = load_c(off_c1, bufs[0], sems_l[0], w=w2[0])
    ld1_b = load_c(off_c1, bufs[0], sems_l[0], w=w2[1])

    @pl.when(cs == 0)
    def _():
        ld1_a.start()

    @pl.when(cs != 0)
    def _():
        ld1_b.start()

    @pl.when(cs == 1)
    def _():
        ld_mt.wait()
        st_mt.start()
        st_mt.wait()

    @pl.when(cs == 0)
    def _():
        ld1_a.wait()
        st = store_c(off_c1, bufs[0], sems_s[0], w=w2[0])
        st.start()
        st.wait()

    @pl.when(cs != 0)
    def _():
        ld1_b.wait()
        st = store_c(off_c1, bufs[0], sems_s[0], w=w2[1])
        st.start()
        st.wait()

    @pl.when(anybits != 0)
    def _():
        def rloop(r, cr):
            rvec = jnp.full((16,), 0, jnp.int32) + r

            @pl.when(((anybits >> r) & 1) != 0)
            def _():
                l0 = plsc.load_gather(
                    tok_v, [rvec, jnp.full((16,), seq - 2, jnp.int32)])[0]
                l1 = plsc.load_gather(
                    tok_v, [rvec, jnp.full((16,), seq - 1, jnp.int32)])[0]

                def sbody(i, c2):
                    idx = lanes + i * 16
                    t0 = plsc.load_gather(tok_v, [rvec, idx])
                    t1 = plsc.load_gather(
                        tok_v, [rvec, jnp.minimum(idx + 1, seq - 1)])
                    t2 = plsc.load_gather(
                        tok_v, [rvec, jnp.minimum(idx + 2, seq - 1)])
                    in_range = ((t2 >= start_col) & (t2 < end_col)) | (
                        (cs == 1) & (t2 >= mt_off))
                    m = (idx < npos) & (t0 == l0) & (t1 == l1) & in_range

                    mi = m.astype(jnp.int32)

                    @pl.when(jnp.any(m))
                    def _():
                        for l in range(16):
                            b = t2[l]

                            @pl.when(mi[l] != 0)
                            def _(b=b, r=r):
                                is_mt = b >= mt_off

                                @pl.when(~is_mt)
                                def _():
                                    colt = pl.multiple_of(
                                        (b >> 7) << 7, 128)
                                    tile = bufs[0].at[:, pl.ds(0, 128)]
                                    cp = pltpu.make_async_copy(
                                        out_hbm.at[pl.ds(row0, _RG),
                                                   pl.ds(colt, 128)],
                                        tile, sem_mt)
                                    cp.start()
                                    cp.wait()
                                    plsc.store_scatter(
                                        tile,
                                        [jnp.full((16,), 0, jnp.int32) + r,
                                         jnp.full((16,), 0, jnp.int32)
                                         + (b - colt)],
                                        neg_inf, mask=lanes == 0)
                                    cp2 = pltpu.make_async_copy(
                                        tile,
                                        out_hbm.at[pl.ds(row0, _RG),
                                                   pl.ds(colt, 128)],
                                        sem_mt)
                                    cp2.start()
                                    cp2.wait()

                                @pl.when(is_mt)
                                def _():
                                    cp = pltpu.make_async_copy(
                                        out_hbm.at[pl.ds(row0, _RG),
                                                   pl.ds(mt_off, _MT)],
                                        mt_v, sem_mt)
                                    cp.start()
                                    cp.wait()
                                    plsc.store_scatter(
                                        mt_v,
                                        [jnp.full((16,), 0, jnp.int32) + r,
                                         jnp.full((16,), 0, jnp.int32)
                                         + (b - mt_off)],
                                        neg_inf, mask=lanes == 0)
                                    cp2 = pltpu.make_async_copy(
                                        mt_v,
                                        out_hbm.at[pl.ds(row0, _RG),
                                                   pl.ds(mt_off, _MT)],
                                        sem_mt)
                                    cp2.start()
                                    cp2.wait()

                    return c2

                lax.fori_loop(0, nmatch, sbody, 0)

            return cr

        lax.fori_loop(0, _RG, rloop, 0)


def kernel(tokens, lprobs, bsz, step, beam_size, no_repeat_ngram_size):
    rows, seq = tokens.shape
    vocab = lprobs.shape[1]
    valid = (
        (rows == bsz * beam_size)
        & (step == seq - 1)
        & (no_repeat_ngram_size == _N)
    )
    valid_arr = jnp.full((16,), 0, dtype=jnp.int32) + valid.astype(jnp.int32)

    mesh = plsc.VectorSubcoreMesh(core_axis_name="c", subcore_axis_name="s")

    def body(valid_hbm, tokens_hbm, lprobs_hbm, out_hbm, tok_v, buf_0,
             mt_v, vld_v, sem_t, sem_l0, sem_s0, sem_mt):
        _body_fn(rows, seq, vocab, valid_hbm, tokens_hbm, lprobs_hbm,
                 out_hbm, tok_v, [buf_0], mt_v, vld_v,
                 sem_t, [sem_l0], [sem_s0], sem_mt)

    run = pl.kernel(
        body,
        out_type=jax.ShapeDtypeStruct((rows, vocab), jnp.float32),
        mesh=mesh,
        compiler_params=pltpu.CompilerParams(
            needs_layout_passes=False,
            skip_device_barrier=True,
        ),
        scratch_types=[
            pltpu.VMEM((_RG, seq), jnp.int32),
            pltpu.VMEM((_RG, _CW), jnp.float32),
            pltpu.VMEM((_RG, _MT), jnp.float32),
            pltpu.VMEM((16,), jnp.int32),
            pltpu.SemaphoreType.DMA,
            pltpu.SemaphoreType.DMA,
            pltpu.SemaphoreType.DMA,
            pltpu.SemaphoreType.DMA,
        ],
    )
    return run(valid_arr, tokens, lprobs)
